# Initial kernel scaffold; baseline (speedup 1.0000x reference)
import functools

import jax
import jax.numpy as jnp
from jax import lax
from jax.experimental import pallas as pl
from jax.experimental.pallas import tpu as pltpu
from jax.experimental.pallas import tpu_sc as plsc

N_NODES = 10000
N_EDGES = 320000
NFEAT = 128
NHID = 128

NC, NS = 2, 16
NW = NC * NS
EPW = N_EDGES // NW
E = 80
NCHUNK = EPW // E
ROWS_PER_TILE = N_NODES // NS
ZROWS = 125
LANES = 16


def _spmm_partials(dst, src, w, x):
    mesh = plsc.VectorSubcoreMesh(
        core_axis_name="c", subcore_axis_name="s", num_cores=NC, num_subcores=NS
    )

    @functools.partial(
        pl.kernel,
        out_type=jax.ShapeDtypeStruct((NC, N_NODES, NFEAT), jnp.float32),
        mesh=mesh,
        scratch_types=[
            pltpu.VMEM((E,), jnp.int32),
            pltpu.VMEM((E,), jnp.int32),
            pltpu.VMEM((E,), jnp.float32),
            pltpu.VMEM((E, NFEAT), jnp.float32),
            pltpu.VMEM((ZROWS, NFEAT), jnp.float32),
            pltpu.VMEM_SHARED((N_NODES, NFEAT), jnp.float32),
            pltpu.SemaphoreType.DMA,
        ],
    )
    def spmm(dst_hbm, src_hbm, w_hbm, x_hbm, out_hbm, sidx, didx, wv, rows,
             zbuf, acc, sem):
        c = lax.axis_index("c")
        s = lax.axis_index("s")
        wid = c * NS + s

        zvec = jnp.zeros((LANES,), jnp.float32)

        def zrow(r, _):
            for j in range(NFEAT // LANES):
                zbuf[r, pl.ds(j * LANES, LANES)] = zvec
            return 0

        lax.fori_loop(0, ZROWS, zrow, 0)
        for k in range(ROWS_PER_TILE // ZROWS):
            pltpu.sync_copy(
                zbuf, acc.at[pl.ds(s * ROWS_PER_TILE + k * ZROWS, ZROWS)]
            )
        plsc.subcore_barrier()

        ebase = wid * EPW

        def chunk(k, _):
            off = ebase + k * E
            pltpu.sync_copy(src_hbm.at[pl.ds(off, E)], sidx)
            pltpu.sync_copy(dst_hbm.at[pl.ds(off, E)], didx)
            pltpu.sync_copy(w_hbm.at[pl.ds(off, E)], wv)
            pltpu.async_copy(x_hbm.at[sidx], rows, sem).wait()

            def scale(e, _):
                wvec = jnp.full((LANES,), wv[e], jnp.float32)
                for j in range(NFEAT // LANES):
                    sl = pl.ds(j * LANES, LANES)
                    rows[e, sl] = rows[e, sl] * wvec
                return 0

            lax.fori_loop(0, E, scale, 0)
            pltpu.sync_copy(rows, acc.at[didx], add=True)
            return 0

        lax.fori_loop(0, NCHUNK, chunk, 0)
        plsc.subcore_barrier()
        pltpu.sync_copy(
            acc.at[pl.ds(s * ROWS_PER_TILE, ROWS_PER_TILE)],
            out_hbm.at[c, pl.ds(s * ROWS_PER_TILE, ROWS_PER_TILE)],
        )

    return spmm(dst, src, w, x)


BM = 1000


def _mm_body(p_ref, w_ref, o_ref):
    agg = p_ref[0] + p_ref[1]
    o_ref[...] = jnp.maximum(
        jnp.dot(agg, w_ref[...], preferred_element_type=jnp.float32), 0.0
    )


def _matmul_relu(partials, W):
    return pl.pallas_call(
        _mm_body,
        grid=(N_NODES // BM,),
        in_specs=[
            pl.BlockSpec((NC, BM, NFEAT), lambda i: (0, i, 0)),
            pl.BlockSpec((NFEAT, NHID), lambda i: (0, 0)),
        ],
        out_specs=pl.BlockSpec((BM, NHID), lambda i: (i, 0)),
        out_shape=jax.ShapeDtypeStruct((N_NODES, NHID), jnp.float32),
    )(partials, W)


def kernel(edge_index, edge_weight, x, W):
    dst = edge_index[0]
    src = edge_index[1]
    partials = _spmm_partials(dst, src, edge_weight, x)
    return _matmul_relu(partials, W)


# speedup vs baseline: 4.5649x; 4.5649x over previous
import functools

import jax
import jax.numpy as jnp
from jax import lax
from jax.experimental import pallas as pl
from jax.experimental.pallas import tpu as pltpu
from jax.experimental.pallas import tpu_sc as plsc

N_NODES = 10000
N_EDGES = 320000
NFEAT = 128
NHID = 128

NC, NS = 2, 16
NW = NC * NS
EPW = N_EDGES // NW
E = 80
NCHUNK = EPW // E
N_PAD = 10240
ROWS_PER_TILE = N_PAD // NS
ZROWS = 128
LANES = 16


def _spmm_partials(dst, src, w, x):
    mesh = plsc.VectorSubcoreMesh(
        core_axis_name="c", subcore_axis_name="s", num_cores=NC, num_subcores=NS
    )

    @functools.partial(
        pl.kernel,
        out_type=jax.ShapeDtypeStruct((NC, N_PAD, NFEAT), jnp.float32),
        mesh=mesh,
        scratch_types=[
            pltpu.VMEM((E,), jnp.int32),
            pltpu.VMEM((E,), jnp.int32),
            pltpu.VMEM((E,), jnp.float32),
            pltpu.VMEM((E, NFEAT), jnp.float32),
            pltpu.VMEM((ZROWS, NFEAT), jnp.float32),
            pltpu.VMEM_SHARED((N_PAD, NFEAT), jnp.float32),
            pltpu.SemaphoreType.DMA,
        ],
    )
    def spmm(dst_hbm, src_hbm, w_hbm, x_hbm, out_hbm, sidx, didx, wv, rows,
             zbuf, acc, sem):
        c = lax.axis_index("c")
        s = lax.axis_index("s")
        wid = c * NS + s

        zvec = jnp.zeros((LANES,), jnp.float32)

        def zrow(r, _):
            for j in range(NFEAT // LANES):
                zbuf[r, pl.ds(j * LANES, LANES)] = zvec
            return 0

        lax.fori_loop(0, ZROWS, zrow, 0)
        for k in range(ROWS_PER_TILE // ZROWS):
            pltpu.sync_copy(
                zbuf, acc.at[pl.ds(s * ROWS_PER_TILE + k * ZROWS, ZROWS)]
            )
        plsc.subcore_barrier()

        ebase = wid * EPW

        def chunk(k, _):
            off = ebase + k * E
            pltpu.sync_copy(src_hbm.at[pl.ds(off, E)], sidx)
            pltpu.sync_copy(dst_hbm.at[pl.ds(off, E)], didx)
            pltpu.sync_copy(w_hbm.at[pl.ds(off, E)], wv)
            pltpu.async_copy(x_hbm.at[sidx], rows, sem).wait()

            def scale(g, _):
                wvec16 = wv[pl.ds(g * LANES, LANES)]
                for el in range(LANES):
                    e = g * LANES + el
                    wsplat = jnp.full((LANES,), wvec16[el], jnp.float32)
                    for j in range(NFEAT // LANES):
                        sl = pl.ds(j * LANES, LANES)
                        rows[e, sl] = rows[e, sl] * wsplat
                return 0

            lax.fori_loop(0, E // LANES, scale, 0)
            pltpu.sync_copy(rows, acc.at[didx], add=True)
            return 0

        lax.fori_loop(0, NCHUNK, chunk, 0)
        plsc.subcore_barrier()
        pltpu.sync_copy(
            acc.at[pl.ds(s * ROWS_PER_TILE, ROWS_PER_TILE)],
            out_hbm.at[c, pl.ds(s * ROWS_PER_TILE, ROWS_PER_TILE)],
        )

    return spmm(dst, src, w, x)


BM = 1000


def _mm_body(p_ref, w_ref, o_ref):
    agg = p_ref[0] + p_ref[1]
    o_ref[...] = jnp.maximum(
        jnp.dot(agg, w_ref[...], preferred_element_type=jnp.float32), 0.0
    )


def _matmul_relu(partials, W):
    return pl.pallas_call(
        _mm_body,
        grid=(N_NODES // BM,),
        in_specs=[
            pl.BlockSpec((NC, BM, NFEAT), lambda i: (0, i, 0)),
            pl.BlockSpec((NFEAT, NHID), lambda i: (0, 0)),
        ],
        out_specs=pl.BlockSpec((BM, NHID), lambda i: (i, 0)),
        out_shape=jax.ShapeDtypeStruct((N_NODES, NHID), jnp.float32),
    )(partials, W)


def kernel(edge_index, edge_weight, x, W):
    dst = edge_index[0]
    src = edge_index[1]
    partials = _spmm_partials(dst, src, edge_weight, x)
    return _matmul_relu(partials, W)


# speedup vs baseline: 8.4599x; 1.8533x over previous
import functools

import jax
import jax.numpy as jnp
from jax import lax
from jax.experimental import pallas as pl
from jax.experimental.pallas import tpu as pltpu
from jax.experimental.pallas import tpu_sc as plsc

N_NODES = 10000
N_EDGES = 320000
NFEAT = 128
NHID = 128

NC, NS = 2, 16
NW = NC * NS
EPW = N_EDGES // NW
E = 80
NCHUNK = EPW // E
N_PAD = 10240
ROWS_PER_TILE = N_PAD // NS
LANES = 16


def _spmm_partials(dst, src, w, x):
    mesh = plsc.VectorSubcoreMesh(
        core_axis_name="c", subcore_axis_name="s", num_cores=NC, num_subcores=NS
    )

    @functools.partial(
        pl.kernel,
        out_type=jax.ShapeDtypeStruct((NC, N_PAD, NFEAT), jnp.float32),
        mesh=mesh,
        scratch_types=[
            [pltpu.VMEM((E, NFEAT), jnp.float32) for _ in range(2)],
            [pltpu.VMEM((E,), jnp.int32) for _ in range(2)],
            [pltpu.VMEM((E,), jnp.int32) for _ in range(2)],
            [pltpu.VMEM((E,), jnp.float32) for _ in range(2)],
            pltpu.VMEM_SHARED((N_PAD, NFEAT), jnp.float32),
            [pltpu.SemaphoreType.DMA for _ in range(2)],
            [pltpu.SemaphoreType.DMA for _ in range(2)],
            [pltpu.SemaphoreType.DMA for _ in range(2)],
        ],
    )
    def spmm(dst_hbm, src_hbm, w_hbm, x_hbm, out_hbm, rows, si, di, wb, acc,
             gsem, ssem, isem):
        c = lax.axis_index("c")
        s = lax.axis_index("s")
        wid = c * NS + s
        ebase = wid * EPW

        def idx_copies(i, b):
            off = ebase + i * E
            return (
                pltpu.async_copy(src_hbm.at[pl.ds(off, E)], si[b], isem[b]),
                pltpu.async_copy(dst_hbm.at[pl.ds(off, E)], di[b], isem[b]),
                pltpu.async_copy(w_hbm.at[pl.ds(off, E)], wb[b], isem[b]),
            )

        def wait_idx(b):
            pltpu.make_async_copy(src_hbm.at[pl.ds(0, E)], si[b], isem[b]).wait()
            pltpu.make_async_copy(dst_hbm.at[pl.ds(0, E)], di[b], isem[b]).wait()
            pltpu.make_async_copy(w_hbm.at[pl.ds(0, E)], wb[b], isem[b]).wait()

        def gather(b):
            pltpu.async_copy(x_hbm.at[si[b]], rows[b], gsem[b])

        def wait_gather(b):
            pltpu.make_async_copy(x_hbm.at[si[b]], rows[b], gsem[b]).wait()

        def scatter(b):
            pltpu.async_copy(rows[b], acc.at[di[b]], ssem[b], add=True)

        def wait_scatter(b):
            pltpu.make_async_copy(rows[b], acc.at[di[b]], ssem[b]).wait()

        def scale(b):
            def grp(g, _):
                wvec16 = wb[b][pl.ds(g * LANES, LANES)]
                for el in range(LANES):
                    wspl = jnp.full((LANES,), wvec16[el], jnp.float32)
                    for j in range(NFEAT // LANES):
                        sl = pl.ds(j * LANES, LANES)
                        e = g * LANES + el
                        rows[b][e, sl] = rows[b][e, sl] * wspl
                return 0

            lax.fori_loop(0, E // LANES, grp, 0)

        zvec = jnp.zeros((LANES,), jnp.float32)

        def zrow(r, _):
            for j in range(NFEAT // LANES):
                rows[0][r, pl.ds(j * LANES, LANES)] = zvec
            return 0

        lax.fori_loop(0, E, zrow, 0)
        for k in range(ROWS_PER_TILE // E):
            pltpu.sync_copy(rows[0], acc.at[pl.ds(s * ROWS_PER_TILE + k * E, E)])
        plsc.subcore_barrier()

        def step(i, cur, first=False, last=False):
            nxt = 1 - cur
            if not first:
                wait_scatter(nxt)
            if not last:
                idx_descs = idx_copies(i + 1, nxt)
            wait_gather(cur)
            scale(cur)
            if not last:
                for d in idx_descs:
                    d.wait()
                gather(nxt)
            scatter(cur)

        for d in idx_copies(0, 0):
            d.wait()
        gather(0)

        step(0, 0, first=True)

        def pair(h, _):
            step(2 * h + 1, 1)
            step(2 * h + 2, 0)
            return 0

        lax.fori_loop(0, (NCHUNK - 3) // 2, pair, 0)

        step(NCHUNK - 2, 1)
        step(NCHUNK - 1, 0, last=True)
        wait_scatter(0)

        plsc.subcore_barrier()
        pltpu.sync_copy(
            acc.at[pl.ds(s * ROWS_PER_TILE, ROWS_PER_TILE)],
            out_hbm.at[c, pl.ds(s * ROWS_PER_TILE, ROWS_PER_TILE)],
        )

    return spmm(dst, src, w, x)


BM = 1000


def _mm_body(p_ref, w_ref, o_ref):
    agg = p_ref[0] + p_ref[1]
    o_ref[...] = jnp.maximum(
        jnp.dot(agg, w_ref[...], preferred_element_type=jnp.float32), 0.0
    )


def _matmul_relu(partials, W):
    return pl.pallas_call(
        _mm_body,
        grid=(N_NODES // BM,),
        in_specs=[
            pl.BlockSpec((NC, BM, NFEAT), lambda i: (0, i, 0)),
            pl.BlockSpec((NFEAT, NHID), lambda i: (0, 0)),
        ],
        out_specs=pl.BlockSpec((BM, NHID), lambda i: (i, 0)),
        out_shape=jax.ShapeDtypeStruct((N_NODES, NHID), jnp.float32),
    )(partials, W)


def kernel(edge_index, edge_weight, x, W):
    dst = edge_index[0]
    src = edge_index[1]
    partials = _spmm_partials(dst, src, edge_weight, x)
    return _matmul_relu(partials, W)
